# baseline (speedup 1.0000x reference)
import functools

import jax
import jax.numpy as jnp
import numpy as np
from jax import lax
from jax.experimental import pallas as pl
from jax.experimental.pallas import tpu as pltpu
from jax.experimental.pallas import tpu_sc as plsc

_M = 32
_N = 2
_SUBSET = 8
_B = 128
_D = 528
_L = 45

_NC, _NS = 2, 16
_NW = _NC * _NS
_BPW = _B // _NW
_PER_W = _BPW * _D
_CH = 128
_NCH = 17
_PAD = _NCH * _CH
_OUTW = 64
_SLOTS = _BPW * _OUTW
_NACC = 16 * _SLOTS


def _fock_states(m, n):
    if n == 0:
        yield (0,) * m
        return
    if m == 1:
        yield (n,)
        return
    for i in range(n + 1):
        for s in _fock_states(m - 1, n - i):
            yield (i,) + s


def _build_tables():
    all_states = list(_fock_states(_M, _N))
    reduced = []
    for i in range(max(0, _SUBSET - _M + _N), _N + 1):
        reduced += list(_fock_states(_SUBSET, i))
    binmap = np.array([reduced.index(s[:_SUBSET]) for s in all_states],
                      dtype=np.int64)

    j = np.arange(_PER_W, dtype=np.int64)
    g = np.arange(_NW, dtype=np.int64)[:, None] * _PER_W + j[None, :]
    b = g // _D
    i = g % _D
    gidx = (b * (_D * _D) + i * (_D + 1)).astype(np.int32)
    gidx = np.concatenate(
        [gidx, np.zeros((_NW, _PAD - _PER_W), np.int32)], axis=1)
    gidx = gidx.reshape(_NW, _NCH, _CH)

    jp = np.arange(_PAD, dtype=np.int64)
    slot = np.where(jp < _PER_W,
                    (jp // _D) * _OUTW + binmap[jp % _D],
                    _OUTW - 1)
    scat = ((jp % 16) * _SLOTS + slot).astype(np.int32)
    return gidx, scat


_GIDX_NP, _SCAT_NP = _build_tables()


def _sc_body(rho_hbm, gidx_hbm, scat_hbm, out_hbm,
             idx_v, scat_v, vals_v, acc_v, stage_v, sem):
    wid = lax.axis_index("s") * _NC + lax.axis_index("c")

    pltpu.sync_copy(gidx_hbm.at[wid], idx_v)
    pltpu.sync_copy(scat_hbm, scat_v)

    copies = [
        pltpu.async_copy(rho_hbm.at[idx_v.at[c]],
                         vals_v.at[pl.ds(c * _CH, _CH)], sem)
        for c in range(_NCH)
    ]

    zeros = jnp.zeros((16,), jnp.float32)

    def _zero_acc(k, carry):
        acc_v[pl.ds(pl.multiple_of(k * 16, 16), 16)] = zeros
        return carry
    lax.fori_loop(0, _NACC // 16, _zero_acc, 0)

    for cp in copies:
        cp.wait()

    def _accum(k, carry):
        off = pl.multiple_of(k * 16, 16)
        tgt = scat_v[pl.ds(off, 16)]
        val = vals_v[pl.ds(off, 16)]
        plsc.addupdate_scatter(acc_v, [tgt], jnp.abs(val))
        return carry
    lax.fori_loop(0, _PAD // 16, _accum, 0)

    def _reduce(k, carry):
        off = pl.multiple_of(k * 16, 16)
        tot = acc_v[pl.ds(off, 16)]
        for cpy in range(1, 16):
            tot = tot + acc_v[pl.ds(cpy * _SLOTS + off, 16)]
        stage_v[pl.ds(off, 16)] = tot
        return carry
    lax.fori_loop(0, _SLOTS // 16, _reduce, 0)

    pltpu.sync_copy(stage_v, out_hbm.at[pl.ds(wid * _SLOTS, _SLOTS)])


@jax.jit
def _partial_measure(rho_flat, gidx, scat):
    mesh = plsc.VectorSubcoreMesh(core_axis_name="c", subcore_axis_name="s",
                                  num_cores=_NC, num_subcores=_NS)
    out = pl.kernel(
        _sc_body,
        out_type=jax.ShapeDtypeStruct((_B * _OUTW,), jnp.float32),
        mesh=mesh,
        compiler_params=pltpu.CompilerParams(needs_layout_passes=False),
        scratch_types=[
            pltpu.VMEM((_NCH, _CH), jnp.int32),
            pltpu.VMEM((_PAD,), jnp.int32),
            pltpu.VMEM((_PAD,), jnp.float32),
            pltpu.VMEM((_NACC,), jnp.float32),
            pltpu.VMEM((_SLOTS,), jnp.float32),
            pltpu.SemaphoreType.DMA,
        ],
    )(rho_flat, gidx, scat)
    return out.reshape(_B, _OUTW)[:, :_L]


def kernel(rho):
    rho_flat = rho.reshape(-1)
    return _partial_measure(rho_flat, jnp.asarray(_GIDX_NP),
                            jnp.asarray(_SCAT_NP))


# speedup vs baseline: 2.1148x; 2.1148x over previous
import jax
import jax.numpy as jnp
import numpy as np
from jax import lax
from jax.experimental import pallas as pl
from jax.experimental.pallas import tpu as pltpu
from jax.experimental.pallas import tpu_sc as plsc

_M = 32
_N = 2
_SUBSET = 8
_B = 128
_D = 528
_L = 45

_NC, _NS = 2, 16
_NW = _NC * _NS
_BPW = _B // _NW
_PER_W = _BPW * _D
_TMAIN = 64
_NBLK = _BPW * _TMAIN
_RING = 64
_OUTW = 64
_SLOTS = _BPW * _OUTW
_NACC = 16 * _SLOTS


def _fock_states(m, n):
    if n == 0:
        yield (0,) * m
        return
    if m == 1:
        yield (n,)
        return
    for i in range(n + 1):
        for s in _fock_states(m - 1, n - i):
            yield (i,) + s


def _build_scatter_table():
    all_states = list(_fock_states(_M, _N))
    reduced = []
    for i in range(max(0, _SUBSET - _M + _N), _N + 1):
        reduced += list(_fock_states(_SUBSET, i))
    binmap = np.array([reduced.index(s[:_SUBSET]) for s in all_states],
                      dtype=np.int64)
    j = np.arange(_PER_W, dtype=np.int64)
    slot = (j // _D) * _OUTW + binmap[j % _D]
    return ((j % 16) * _SLOTS + slot).astype(np.int32)


_SCAT_NP = _build_scatter_table()


def _sc_body(rho_hbm, tail_hbm, scat_hbm, out_hbm,
             buf_v, tail_v, scat_v, acc_v, stage_v, sem, tsem):
    wid = lax.axis_index("s") * _NC + lax.axis_index("c")
    row0 = wid * (_BPW * _D)
    lane = lax.iota(jnp.int32, 16)

    for bl in range(_BPW):
        tidx = (wid * _BPW + bl) * 256 + lane * 17
        pltpu.async_copy(tail_hbm.at[tidx],
                         tail_v.at[pl.ds(bl * 16, 16)], tsem)

    pltpu.sync_copy(scat_hbm, scat_v)

    def _issue(g):
        bl = g // _TMAIN
        t = g - bl * _TMAIN
        r = row0 + bl * _D + 8 * t
        c = (t // 16) * 128
        slot_row = pl.multiple_of((g % _RING) * 8, 8)
        pltpu.async_copy(rho_hbm.at[pl.ds(r, 8), pl.ds(c, 128)],
                         buf_v.at[pl.ds(slot_row, 8)], sem)

    def _prologue(g, carry):
        _issue(g)
        return carry
    lax.fori_loop(0, _RING, _prologue, 0)

    zeros = jnp.zeros((16,), jnp.float32)

    def _zero_acc(k, carry):
        acc_v[pl.ds(pl.multiple_of(k * 16, 16), 16)] = zeros
        return carry
    lax.fori_loop(0, _NACC // 16, _zero_acc, 0)

    def _step(u, carry):
        g = 2 * u
        base = pl.multiple_of((g % _RING) * 8, 16)
        pltpu.make_async_copy(rho_hbm.at[pl.ds(0, 16), pl.ds(0, 128)],
                              buf_v.at[pl.ds(base, 16)], sem).wait()
        p = u % (_TMAIN // 2)
        cbase = (16 * p) % 128
        vals = plsc.load_gather(buf_v, [base + lane, cbase + lane])
        bl = u // (_TMAIN // 2)
        joff = pl.multiple_of(bl * _D + 16 * p, 16)
        tgt = scat_v[pl.ds(joff, 16)]
        plsc.addupdate_scatter(acc_v, [tgt], jnp.abs(vals))

        @pl.when(g + _RING < _NBLK)
        def _():
            _issue(g + _RING)

        @pl.when(g + 1 + _RING < _NBLK)
        def _():
            _issue(g + 1 + _RING)
        return carry
    lax.fori_loop(0, _NBLK // 2, _step, 0)

    for bl in range(_BPW):
        pltpu.make_async_copy(tail_hbm.at[pl.ds(0, 16)],
                              tail_v.at[pl.ds(bl * 16, 16)], tsem).wait()
    for bl in range(_BPW):
        vals = tail_v[pl.ds(bl * 16, 16)]
        tgt = scat_v[pl.ds(bl * _D + 512, 16)]
        plsc.addupdate_scatter(acc_v, [tgt], jnp.abs(vals))

    def _reduce(k, carry):
        off = pl.multiple_of(k * 16, 16)
        tot = acc_v[pl.ds(off, 16)]
        for cpy in range(1, 16):
            tot = tot + acc_v[pl.ds(cpy * _SLOTS + off, 16)]
        stage_v[pl.ds(off, 16)] = tot
        return carry
    lax.fori_loop(0, _SLOTS // 16, _reduce, 0)

    pltpu.sync_copy(stage_v, out_hbm.at[pl.ds(wid * _SLOTS, _SLOTS)])


@jax.jit
def _partial_measure(rho2d, tail_lin, scat):
    mesh = plsc.VectorSubcoreMesh(core_axis_name="c", subcore_axis_name="s",
                                  num_cores=_NC, num_subcores=_NS)
    out = pl.kernel(
        _sc_body,
        out_type=jax.ShapeDtypeStruct((_B * _OUTW,), jnp.float32),
        mesh=mesh,
        compiler_params=pltpu.CompilerParams(needs_layout_passes=False,
                                             use_tc_tiling_on_sc=True),
        scratch_types=[
            pltpu.VMEM((_RING * 8, 128), jnp.float32),
            pltpu.VMEM((_BPW * 16,), jnp.float32),
            pltpu.VMEM((_PER_W,), jnp.int32),
            pltpu.VMEM((_NACC,), jnp.float32),
            pltpu.VMEM((_SLOTS,), jnp.float32),
            pltpu.SemaphoreType.DMA,
            pltpu.SemaphoreType.DMA,
        ],
    )(rho2d, tail_lin, scat)
    return out.reshape(_B, _OUTW)[:, :_L]


def kernel(rho):
    rho2d = rho.reshape(_B * _D, _D)
    tail_lin = rho[:, 512:, 512:].reshape(-1)
    return _partial_measure(rho2d, tail_lin, jnp.asarray(_SCAT_NP))


# speedup vs baseline: 10.5671x; 4.9967x over previous
import jax
import jax.numpy as jnp
import numpy as np
from jax import lax
from jax.experimental import pallas as pl
from jax.experimental.pallas import tpu as pltpu
from jax.experimental.pallas import tpu_sc as plsc

_M = 32
_N = 2
_SUBSET = 8
_B = 128
_D = 528
_L = 45

_NC, _NS = 2, 16
_NW = _NC * _NS
_NRC = 4
_NBC = 8
_RPC = _D // _NRC
_RPAD = 136
_NBIN = _L + 1
_ACC = _NBIN * 16


def _fock_states(m, n):
    if n == 0:
        yield (0,) * m
        return
    if m == 1:
        yield (n,)
        return
    for i in range(n + 1):
        for s in _fock_states(m - 1, n - i):
            yield (i,) + s


def _build_tables():
    all_states = list(_fock_states(_M, _N))
    reduced = []
    for i in range(max(0, _SUBSET - _M + _N), _N + 1):
        reduced += list(_fock_states(_SUBSET, i))
    binmap = np.array([reduced.index(s[:_SUBSET]) for s in all_states],
                      dtype=np.int64)
    gidx = np.zeros((_NRC * _RPAD,), np.int32)
    bins = np.full((_NRC * _RPAD,), _L, np.int32)
    for c in range(_NRC):
        for t in range(_RPC):
            i = c * _RPC + t
            gidx[c * _RPAD + t] = (_D + 1) * i
            bins[c * _RPAD + t] = binmap[i]
    return gidx, bins


_GIDX_NP, _BINS_NP = _build_tables()


def _sc_body(diag_hbm, gidx_hbm, bins_hbm, out_hbm,
             idx_v, bins_v, vals_v, acc_v, sem):
    wid = lax.axis_index("s") * _NC + lax.axis_index("c")
    rc = wid // _NBC
    bc = wid - rc * _NBC
    boff = pl.multiple_of(bc * 16, 16)

    toff = pl.multiple_of(rc * _RPAD, 8)
    pltpu.sync_copy(gidx_hbm.at[pl.ds(toff, _RPAD)], idx_v)
    pltpu.sync_copy(bins_hbm.at[pl.ds(toff, _RPAD)], bins_v.at[pl.ds(0, _RPAD)])

    cp0 = pltpu.async_copy(diag_hbm.at[idx_v.at[pl.ds(0, 64)]],
                           vals_v.at[pl.ds(0, 64)], sem)
    cp1 = pltpu.async_copy(diag_hbm.at[idx_v.at[pl.ds(64, 72)]],
                           vals_v.at[pl.ds(64, 72)], sem)

    zeros = jnp.zeros((16,), jnp.float32)

    def _zero_acc(k, carry):
        acc_v[pl.ds(pl.multiple_of(k * 16, 16), 16)] = zeros
        return carry
    lax.fori_loop(0, _NBIN, _zero_acc, 0)

    cp0.wait()
    cp1.wait()

    def _accum(t, carry):
        b = bins_v[pl.ds(t, 16)][0]
        val = vals_v[t, pl.ds(boff, 16)]
        off = pl.multiple_of(b * 16, 16)
        acc_v[pl.ds(off, 16)] = acc_v[pl.ds(off, 16)] + jnp.abs(val)
        return carry
    lax.fori_loop(0, _RPAD, _accum, 0)

    pltpu.sync_copy(acc_v, out_hbm.at[pl.ds(wid * _ACC, _ACC)])


@jax.jit
def _partial_measure(diag_tab, gidx, bins):
    mesh = plsc.VectorSubcoreMesh(core_axis_name="c", subcore_axis_name="s",
                                  num_cores=_NC, num_subcores=_NS)
    parts = pl.kernel(
        _sc_body,
        out_type=jax.ShapeDtypeStruct((_NW * _ACC,), jnp.float32),
        mesh=mesh,
        compiler_params=pltpu.CompilerParams(needs_layout_passes=False,
                                             use_tc_tiling_on_sc=True),
        scratch_types=[
            pltpu.VMEM((_RPAD,), jnp.int32),
            pltpu.VMEM((_RPAD + 16,), jnp.int32),
            pltpu.VMEM((_RPAD, _B), jnp.float32),
            pltpu.VMEM((_ACC,), jnp.float32),
            pltpu.SemaphoreType.DMA,
        ],
    )(diag_tab, gidx, bins)
    p = parts.reshape(_NRC, _NBC, _NBIN, 16).sum(0)
    return p.transpose(0, 2, 1).reshape(_B, _NBIN)[:, :_L]


def kernel(rho):
    diag_tab = jnp.transpose(rho, (1, 2, 0)).reshape(_D * _D, _B)
    return _partial_measure(diag_tab, jnp.asarray(_GIDX_NP),
                            jnp.asarray(_BINS_NP))


# speedup vs baseline: 12.0851x; 1.1437x over previous
import jax
import jax.numpy as jnp
import numpy as np
from jax import lax
from jax.experimental import pallas as pl
from jax.experimental.pallas import tpu as pltpu
from jax.experimental.pallas import tpu_sc as plsc

_M = 32
_N = 2
_SUBSET = 8
_B = 128
_D = 528
_L = 45

_NC, _NS = 2, 16
_NW = _NC * _NS
_NRC = 4
_NBC = 8
_RPC = _D // _NRC
_ACC = _L * 16


def _fock_states(m, n):
    if n == 0:
        yield (0,) * m
        return
    if m == 1:
        yield (n,)
        return
    for i in range(n + 1):
        for s in _fock_states(m - 1, n - i):
            yield (i,) + s


def _build_binmap():
    all_states = list(_fock_states(_M, _N))
    reduced = []
    for i in range(max(0, _SUBSET - _M + _N), _N + 1):
        reduced += list(_fock_states(_SUBSET, i))
    return [reduced.index(s[:_SUBSET]) for s in all_states]


_BINMAP = _build_binmap()

_CHUNK_BASES = [16 * k for k in range(8)] + [116]

_GROUPS = []
for _c in range(_NRC):
    g = {}
    for _r in range(_RPC):
        _p = _r if _r < 128 else _r + 12
        g.setdefault(_BINMAP[_c * _RPC + _r], []).append(_p)
    _GROUPS.append(g)


def _sc_body(diag_hbm, out_hbm, vals_v, stage_v, sem):
    wid = lax.axis_index("s") * _NC + lax.axis_index("c")
    rc = wid // _NBC
    bc = wid - rc * _NBC
    boff = pl.multiple_of(bc * 16, 16)

    lane = lax.iota(jnp.int32, 16)
    base = rc * ((_D + 1) * _RPC)
    copies = []
    for k, cb in enumerate(_CHUNK_BASES):
        idx = base + ((_D + 1) * cb + (_D + 1) * lane)
        copies.append(
            pltpu.async_copy(diag_hbm.at[idx],
                             vals_v.at[pl.ds(16 * k, 16)], sem))
    for cp in copies:
        cp.wait()

    zeros = jnp.zeros((16,), jnp.float32)
    for c in range(_NRC):
        @pl.when(rc == c)
        def _(c=c):
            for b in range(_L):
                rows = _GROUPS[c].get(b)
                if rows is None:
                    stage_v[pl.ds(b * 16, 16)] = zeros
                    continue
                tot = jnp.abs(vals_v[rows[0], pl.ds(boff, 16)])
                for p in rows[1:]:
                    tot = tot + jnp.abs(vals_v[p, pl.ds(boff, 16)])
                stage_v[pl.ds(b * 16, 16)] = tot

    pltpu.sync_copy(stage_v, out_hbm.at[pl.ds(wid * _ACC, _ACC)])


@jax.jit
def _partial_measure(diag_tab):
    mesh = plsc.VectorSubcoreMesh(core_axis_name="c", subcore_axis_name="s",
                                  num_cores=_NC, num_subcores=_NS)
    parts = pl.kernel(
        _sc_body,
        out_type=jax.ShapeDtypeStruct((_NW * _ACC,), jnp.float32),
        mesh=mesh,
        compiler_params=pltpu.CompilerParams(needs_layout_passes=False,
                                             use_tc_tiling_on_sc=True),
        scratch_types=[
            pltpu.VMEM((144, _B), jnp.float32),
            pltpu.VMEM((_ACC,), jnp.float32),
            pltpu.SemaphoreType.DMA,
        ],
    )(diag_tab)
    p = parts.reshape(_NRC, _NBC, _L, 16).sum(0)
    return p.transpose(0, 2, 1).reshape(_B, _L)


def kernel(rho):
    diag_tab = jnp.transpose(rho, (1, 2, 0)).reshape(_D * _D, _B)
    return _partial_measure(diag_tab)
